# baseline (device time: 112456 ns/iter reference)
import jax
import jax.numpy as jnp
from jax import lax
from jax.experimental import pallas as pl
from jax.experimental.pallas import tpu as pltpu

N_DEV = 8
M = 1024
N = 1024
CHUNK = M // N_DEV
N_HOPS = 2 * (N_DEV - 1)


def kernel(x, w_mat):
    m, k = x.shape
    k2, n = w_mat.shape

    def body(x_ref, w_ref, out_ref, comm_ref, send_sems, recv_sems):
        my = lax.axis_index("i")
        left = (my - 1 + N_DEV) % N_DEV
        right = (my + 1) % N_DEV

        barrier_sem = pltpu.get_barrier_semaphore()
        for nbr in (left, right):
            pl.semaphore_signal(
                barrier_sem, inc=1,
                device_id=(nbr,), device_id_type=pl.DeviceIdType.MESH,
            )
        pl.semaphore_wait(barrier_sem, 2)

        out_ref[...] = jnp.dot(
            x_ref[...], w_ref[...], preferred_element_type=jnp.float32
        )

        def chunk(c):
            return pl.ds(c * CHUNK, CHUNK)

        for s in range(N_DEV - 1):
            send_c = (my - s + N_DEV) % N_DEV
            rdma = pltpu.make_async_remote_copy(
                src_ref=out_ref.at[chunk(send_c), :],
                dst_ref=comm_ref.at[s],
                send_sem=send_sems.at[s],
                recv_sem=recv_sems.at[s],
                device_id=(right,),
                device_id_type=pl.DeviceIdType.MESH,
            )
            rdma.start()
            rdma.wait()
            recv_c = (my - s - 1 + N_DEV) % N_DEV
            out_ref[chunk(recv_c), :] = (
                out_ref[chunk(recv_c), :] + comm_ref[s, :, :]
            )

        for t in range(N_DEV - 1):
            h = (N_DEV - 1) + t
            send_c = (my + 1 - t + N_DEV) % N_DEV
            rdma = pltpu.make_async_remote_copy(
                src_ref=out_ref.at[chunk(send_c), :],
                dst_ref=comm_ref.at[h],
                send_sem=send_sems.at[h],
                recv_sem=recv_sems.at[h],
                device_id=(right,),
                device_id_type=pl.DeviceIdType.MESH,
            )
            rdma.start()
            rdma.wait()
            recv_c = (my - t + N_DEV) % N_DEV
            out_ref[chunk(recv_c), :] = comm_ref[h, :, :]

    return pl.pallas_call(
        body,
        out_shape=jax.ShapeDtypeStruct((M, N), jnp.float32),
        in_specs=[
            pl.BlockSpec(memory_space=pltpu.VMEM),
            pl.BlockSpec(memory_space=pltpu.VMEM),
        ],
        out_specs=pl.BlockSpec(memory_space=pltpu.VMEM),
        scratch_shapes=[
            pltpu.VMEM((N_HOPS, CHUNK, N), jnp.float32),
            pltpu.SemaphoreType.DMA((N_HOPS,)),
            pltpu.SemaphoreType.DMA((N_HOPS,)),
        ],
        compiler_params=pltpu.CompilerParams(collective_id=0),
    )(x, w_mat)


# device time: 49371 ns/iter; 2.2778x vs baseline; 2.2778x over previous
import jax
import jax.numpy as jnp
from jax import lax
from jax.experimental import pallas as pl
from jax.experimental.pallas import tpu as pltpu

M = 1024
N = 1024

PARTS = (
    (0, 384, ("x", "y", "z")),
    (384, 384, ("y", "z", "x")),
    (768, 256, ("z", "x", "y")),
)
RS_OFF = (0, 512, 768)
AG_OFF = (896, 1024, 1280)


def kernel(x, w_mat):
    def body(x_ref, w_ref, out_ref, comm0, comm1, comm2, send_sems, recv_sems):
        comms = (comm0, comm1, comm2)

        my = lax.axis_index("i")
        r4 = my % 4
        bz = my // 4
        bx = (r4 ^ (r4 >> 1)) & 1
        by = r4 // 2

        def pos_of(tx, ty, tz):
            return 4 * tz + 2 * ty + (tx ^ ty)

        partner = {
            "x": pos_of(1 - bx, by, bz),
            "y": pos_of(bx, 1 - by, bz),
            "z": pos_of(bx, by, 1 - bz),
        }
        bit = {"x": bx, "y": by, "z": bz}

        barrier_sem = pltpu.get_barrier_semaphore()
        for ax in ("x", "y", "z"):
            pl.semaphore_signal(
                barrier_sem, inc=1,
                device_id=(partner[ax],), device_id_type=pl.DeviceIdType.MESH,
            )
        pl.semaphore_wait(barrier_sem, 3)

        out_ref[...] = jnp.dot(
            x_ref[...], w_ref[...], preferred_element_type=jnp.float32
        )

        rstart = [jnp.int32(0), jnp.int32(0), jnp.int32(0)]

        def dsrow(start, size):
            return pl.ds(pl.multiple_of(start, 128), size)

        for k in range(3):
            half = 512 >> k
            rdmas = []
            for p, (c0, w, order) in enumerate(PARTS):
                ax = order[k]
                b = bit[ax]
                send_start = rstart[p] + (1 - b) * half
                rdma = pltpu.make_async_remote_copy(
                    src_ref=out_ref.at[dsrow(send_start, half), pl.ds(c0, w)],
                    dst_ref=comms[p].at[pl.ds(RS_OFF[k], half), :],
                    send_sem=send_sems.at[p * 6 + k],
                    recv_sem=recv_sems.at[p * 6 + k],
                    device_id=(partner[ax],),
                    device_id_type=pl.DeviceIdType.MESH,
                )
                rdma.start()
                rdmas.append(rdma)
            for p, (c0, w, order) in enumerate(PARTS):
                ax = order[k]
                b = bit[ax]
                keep = rstart[p] + b * half
                rdmas[p].wait()
                out_ref[dsrow(keep, half), c0:c0 + w] = (
                    out_ref[dsrow(keep, half), c0:c0 + w]
                    + comms[p][RS_OFF[k]:RS_OFF[k] + half, :]
                )
                rstart[p] = keep

        for j, k in enumerate((2, 1, 0)):
            cur = 128 << j
            rdmas = []
            for p, (c0, w, order) in enumerate(PARTS):
                ax = order[k]
                rdma = pltpu.make_async_remote_copy(
                    src_ref=out_ref.at[dsrow(rstart[p], cur), pl.ds(c0, w)],
                    dst_ref=comms[p].at[pl.ds(AG_OFF[j], cur), :],
                    send_sem=send_sems.at[p * 6 + 3 + j],
                    recv_sem=recv_sems.at[p * 6 + 3 + j],
                    device_id=(partner[ax],),
                    device_id_type=pl.DeviceIdType.MESH,
                )
                rdma.start()
                rdmas.append(rdma)
            for p, (c0, w, order) in enumerate(PARTS):
                ax = order[k]
                b = bit[ax]
                partner_start = rstart[p] + cur - 2 * b * cur
                rdmas[p].wait()
                out_ref[dsrow(partner_start, cur), c0:c0 + w] = comms[p][
                    AG_OFF[j]:AG_OFF[j] + cur, :
                ]
                rstart[p] = jnp.minimum(rstart[p], partner_start)

    return pl.pallas_call(
        body,
        out_shape=jax.ShapeDtypeStruct((M, N), jnp.float32),
        in_specs=[
            pl.BlockSpec(memory_space=pltpu.VMEM),
            pl.BlockSpec(memory_space=pltpu.VMEM),
        ],
        out_specs=pl.BlockSpec(memory_space=pltpu.VMEM),
        scratch_shapes=[
            pltpu.VMEM((1792, 384), jnp.float32),
            pltpu.VMEM((1792, 384), jnp.float32),
            pltpu.VMEM((1792, 256), jnp.float32),
            pltpu.SemaphoreType.DMA((18,)),
            pltpu.SemaphoreType.DMA((18,)),
        ],
        compiler_params=pltpu.CompilerParams(collective_id=0),
    )(x, w_mat)


# device time: 48856 ns/iter; 2.3018x vs baseline; 1.0105x over previous
import jax
import jax.numpy as jnp
from jax import lax
from jax.experimental import pallas as pl
from jax.experimental.pallas import tpu as pltpu

M = 1024
N = 1024

PARTS = (
    (0, 384, ("x", "y", "z")),
    (384, 384, ("y", "z", "x")),
    (768, 256, ("z", "x", "y")),
)
RS_OFF = (0, 512, 768)


def kernel(x, w_mat):
    def body(x_ref, w_ref, out_ref, comm0, comm1, comm2, send_sems, recv_sems):
        comms = (comm0, comm1, comm2)

        my = lax.axis_index("i")
        r4 = my % 4
        bz = my // 4
        bx = (r4 ^ (r4 >> 1)) & 1
        by = r4 // 2

        def pos_of(tx, ty, tz):
            return 4 * tz + 2 * ty + (tx ^ ty)

        partner = {
            "x": pos_of(1 - bx, by, bz),
            "y": pos_of(bx, 1 - by, bz),
            "z": pos_of(bx, by, 1 - bz),
        }
        bit = {"x": bx, "y": by, "z": bz}

        barrier_sem = pltpu.get_barrier_semaphore()
        for ax in ("x", "y", "z"):
            pl.semaphore_signal(
                barrier_sem, inc=1,
                device_id=(partner[ax],), device_id_type=pl.DeviceIdType.MESH,
            )
        pl.semaphore_wait(barrier_sem, 3)

        out_ref[...] = jnp.dot(
            x_ref[...], w_ref[...], preferred_element_type=jnp.float32
        )

        def dsrow(start, size):
            return pl.ds(pl.multiple_of(start, 128), size)

        rstart = [jnp.int32(0), jnp.int32(0), jnp.int32(0)]

        def start_rs(p, k):
            c0, w, order = PARTS[p]
            half = 512 >> k
            b = bit[order[k]]
            send_start = rstart[p] + (1 - b) * half
            rdma = pltpu.make_async_remote_copy(
                src_ref=out_ref.at[dsrow(send_start, half), pl.ds(c0, w)],
                dst_ref=comms[p].at[pl.ds(RS_OFF[k], half), :],
                send_sem=send_sems.at[p * 6 + k],
                recv_sem=recv_sems.at[p * 6 + k],
                device_id=(partner[order[k]],),
                device_id_type=pl.DeviceIdType.MESH,
            )
            rdma.start()
            return rdma

        def finish_rs(p, k):
            c0, w, order = PARTS[p]
            half = 512 >> k
            keep = rstart[p] + bit[order[k]] * half
            out_ref[dsrow(keep, half), c0:c0 + w] = (
                out_ref[dsrow(keep, half), c0:c0 + w]
                + comms[p][RS_OFF[k]:RS_OFF[k] + half, :]
            )
            rstart[p] = keep

        def start_ag(p, j):
            c0, w, order = PARTS[p]
            cur = 128 << j
            ax = order[2 - j]
            rdma = pltpu.make_async_remote_copy(
                src_ref=out_ref.at[dsrow(rstart[p], cur), pl.ds(c0, w)],
                dst_ref=out_ref.at[dsrow(rstart[p], cur), pl.ds(c0, w)],
                send_sem=send_sems.at[p * 6 + 3 + j],
                recv_sem=recv_sems.at[p * 6 + 3 + j],
                device_id=(partner[ax],),
                device_id_type=pl.DeviceIdType.MESH,
            )
            rdma.start()
            return rdma

        def merge_ag(p, j):
            _, _, order = PARTS[p]
            cur = 128 << j
            b = bit[order[2 - j]]
            rstart[p] = rstart[p] - b * cur

        rdmas = [start_rs(p, 0) for p in range(3)]
        for k in (1, 2):
            for p in range(3):
                rdmas[p].wait()
                finish_rs(p, k - 1)
                rdmas[p] = start_rs(p, k)
        for p in range(3):
            rdmas[p].wait()
            finish_rs(p, 2)
            rdmas[p] = start_ag(p, 0)
        for j in (1, 2):
            for p in range(3):
                rdmas[p].wait()
                merge_ag(p, j - 1)
                rdmas[p] = start_ag(p, j)
        for p in range(3):
            rdmas[p].wait()
            merge_ag(p, 2)

    return pl.pallas_call(
        body,
        out_shape=jax.ShapeDtypeStruct((M, N), jnp.float32),
        in_specs=[
            pl.BlockSpec(memory_space=pltpu.VMEM),
            pl.BlockSpec(memory_space=pltpu.VMEM),
        ],
        out_specs=pl.BlockSpec(memory_space=pltpu.VMEM),
        scratch_shapes=[
            pltpu.VMEM((896, 384), jnp.float32),
            pltpu.VMEM((896, 384), jnp.float32),
            pltpu.VMEM((896, 256), jnp.float32),
            pltpu.SemaphoreType.DMA((18,)),
            pltpu.SemaphoreType.DMA((18,)),
        ],
        compiler_params=pltpu.CompilerParams(collective_id=0),
    )(x, w_mat)


# device time: 42662 ns/iter; 2.6360x vs baseline; 1.1452x over previous
import jax
import jax.numpy as jnp
from jax import lax
from jax.experimental import pallas as pl
from jax.experimental.pallas import tpu as pltpu

M = 1024
N = 1024

ORDERS = (("x", "y", "z"), ("y", "z", "x"), ("z", "x", "y"))
CHAINS = (
    (0, 168, 0),
    (168, 176, 0),
    (344, 168, 1),
    (512, 176, 1),
    (688, 168, 2),
    (856, 168, 2),
)
BAND_ROWS = ((0, 344), (344, 344), (688, 336))
RS_OFF = (0, 512, 768)


def kernel(x, w_mat):
    def body(x_ref, w_ref, out_ref, c0, c1, c2, c3, c4, c5, send_sems, recv_sems):
        comms = (c0, c1, c2, c3, c4, c5)

        my = lax.axis_index("i")
        r4 = my % 4
        bz = my // 4
        bx = (r4 ^ (r4 >> 1)) & 1
        by = r4 // 2

        def pos_of(tx, ty, tz):
            return 4 * tz + 2 * ty + (tx ^ ty)

        partner = {
            "x": pos_of(1 - bx, by, bz),
            "y": pos_of(bx, 1 - by, bz),
            "z": pos_of(bx, by, 1 - bz),
        }
        bit = {"x": bx, "y": by, "z": bz}

        barrier_sem = pltpu.get_barrier_semaphore()
        for ax in ("x", "y", "z"):
            pl.semaphore_signal(
                barrier_sem, inc=1,
                device_id=(partner[ax],), device_id_type=pl.DeviceIdType.MESH,
            )
        pl.semaphore_wait(barrier_sem, 3)

        def dscol(start, size):
            return pl.ds(pl.multiple_of(start, 128), size)

        cstart = [jnp.int32(0) for _ in CHAINS]

        def start_rs(ci, k):
            r0, rl, band = CHAINS[ci]
            half = 512 >> k
            ax = ORDERS[band][k]
            send_c = cstart[ci] + (1 - bit[ax]) * half
            rdma = pltpu.make_async_remote_copy(
                src_ref=out_ref.at[pl.ds(r0, rl), dscol(send_c, half)],
                dst_ref=comms[ci].at[:, pl.ds(RS_OFF[k], half)],
                send_sem=send_sems.at[ci * 6 + k],
                recv_sem=recv_sems.at[ci * 6 + k],
                device_id=(partner[ax],),
                device_id_type=pl.DeviceIdType.MESH,
            )
            rdma.start()
            return rdma

        def finish_rs(ci, k):
            r0, rl, band = CHAINS[ci]
            half = 512 >> k
            keep = cstart[ci] + bit[ORDERS[band][k]] * half
            out_ref[pl.ds(r0, rl), dscol(keep, half)] = (
                out_ref[pl.ds(r0, rl), dscol(keep, half)]
                + comms[ci][:, RS_OFF[k]:RS_OFF[k] + half]
            )
            cstart[ci] = keep

        def start_ag(ci, j):
            r0, rl, band = CHAINS[ci]
            cur = 128 << j
            ax = ORDERS[band][2 - j]
            rdma = pltpu.make_async_remote_copy(
                src_ref=out_ref.at[pl.ds(r0, rl), dscol(cstart[ci], cur)],
                dst_ref=out_ref.at[pl.ds(r0, rl), dscol(cstart[ci], cur)],
                send_sem=send_sems.at[ci * 6 + 3 + j],
                recv_sem=recv_sems.at[ci * 6 + 3 + j],
                device_id=(partner[ax],),
                device_id_type=pl.DeviceIdType.MESH,
            )
            rdma.start()
            return rdma

        def merge_ag(ci, j):
            _, _, band = CHAINS[ci]
            cur = 128 << j
            cstart[ci] = cstart[ci] - bit[ORDERS[band][2 - j]] * cur

        rdmas = [None] * len(CHAINS)
        for band, (br0, brl) in enumerate(BAND_ROWS):
            out_ref[br0:br0 + brl, :] = jnp.dot(
                x_ref[br0:br0 + brl, :], w_ref[...],
                preferred_element_type=jnp.float32,
            )
            for ci, (_, _, b) in enumerate(CHAINS):
                if b == band:
                    rdmas[ci] = start_rs(ci, 0)

        for k in (1, 2):
            for ci in range(len(CHAINS)):
                rdmas[ci].wait()
                finish_rs(ci, k - 1)
                rdmas[ci] = start_rs(ci, k)
        for ci in range(len(CHAINS)):
            rdmas[ci].wait()
            finish_rs(ci, 2)
            rdmas[ci] = start_ag(ci, 0)
        for j in (1, 2):
            for ci in range(len(CHAINS)):
                rdmas[ci].wait()
                merge_ag(ci, j - 1)
                rdmas[ci] = start_ag(ci, j)
        for ci in range(len(CHAINS)):
            rdmas[ci].wait()
            merge_ag(ci, 2)

    n_sems = 6 * len(CHAINS)
    return pl.pallas_call(
        body,
        out_shape=jax.ShapeDtypeStruct((M, N), jnp.float32),
        in_specs=[
            pl.BlockSpec(memory_space=pltpu.VMEM),
            pl.BlockSpec(memory_space=pltpu.VMEM),
        ],
        out_specs=pl.BlockSpec(memory_space=pltpu.VMEM),
        scratch_shapes=[
            pltpu.VMEM((rl, 896), jnp.float32) for (_, rl, _) in CHAINS
        ] + [
            pltpu.SemaphoreType.DMA((n_sems,)),
            pltpu.SemaphoreType.DMA((n_sems,)),
        ],
        compiler_params=pltpu.CompilerParams(collective_id=0),
    )(x, w_mat)


# device time: 42111 ns/iter; 2.6705x vs baseline; 1.0131x over previous
import jax
import jax.numpy as jnp
from jax import lax
from jax.experimental import pallas as pl
from jax.experimental.pallas import tpu as pltpu

M = 1024
N = 1024

ORDERS = (("x", "y", "z"), ("y", "z", "x"), ("z", "x", "y"))
CHAINS = (
    (0, 168, 0),
    (168, 176, 0),
    (344, 168, 1),
    (512, 176, 1),
    (688, 168, 2),
    (856, 168, 2),
)
BAND_ROWS = ((0, 344), (344, 344), (688, 336))
RS_OFF = (0, 512, 768)
SEMS_PER_CHAIN = 5


def kernel(x, w_mat):
    def body(x_ref, w_ref, out_ref, c0, c1, c2, c3, c4, c5, send_sems, recv_sems):
        comms = (c0, c1, c2, c3, c4, c5)

        my = lax.axis_index("i")
        r4 = my % 4
        bz = my // 4
        bx = (r4 ^ (r4 >> 1)) & 1
        by = r4 // 2

        def pos_of(tx, ty, tz):
            return 4 * tz + 2 * ty + (tx ^ ty)

        partner = {
            "x": pos_of(1 - bx, by, bz),
            "y": pos_of(bx, 1 - by, bz),
            "z": pos_of(bx, by, 1 - bz),
        }
        bit = {"x": bx, "y": by, "z": bz}

        barrier_sem = pltpu.get_barrier_semaphore()
        for ax in ("x", "y", "z"):
            pl.semaphore_signal(
                barrier_sem, inc=1,
                device_id=(partner[ax],), device_id_type=pl.DeviceIdType.MESH,
            )
        pl.semaphore_wait(barrier_sem, 3)

        def dscol(start, size):
            return pl.ds(pl.multiple_of(start, 128), size)

        cstart = [jnp.int32(0) for _ in CHAINS]

        def start_rs(ci, k):
            r0, rl, band = CHAINS[ci]
            half = 512 >> k
            ax = ORDERS[band][k]
            send_c = cstart[ci] + (1 - bit[ax]) * half
            rdma = pltpu.make_async_remote_copy(
                src_ref=out_ref.at[pl.ds(r0, rl), dscol(send_c, half)],
                dst_ref=comms[ci].at[:, pl.ds(RS_OFF[k], half)],
                send_sem=send_sems.at[ci * SEMS_PER_CHAIN + k],
                recv_sem=recv_sems.at[ci * SEMS_PER_CHAIN + k],
                device_id=(partner[ax],),
                device_id_type=pl.DeviceIdType.MESH,
            )
            rdma.start()
            return rdma

        def finish_rs(ci, k):
            r0, rl, band = CHAINS[ci]
            half = 512 >> k
            keep = cstart[ci] + bit[ORDERS[band][k]] * half
            out_ref[pl.ds(r0, rl), dscol(keep, half)] = (
                out_ref[pl.ds(r0, rl), dscol(keep, half)]
                + comms[ci][:, RS_OFF[k]:RS_OFF[k] + half]
            )
            cstart[ci] = keep

        def start_leaf(ci):
            r0, rl, band = CHAINS[ci]
            ax = ORDERS[band][2]
            rdma = pltpu.make_async_remote_copy(
                src_ref=out_ref.at[pl.ds(r0, rl), dscol(cstart[ci], 256)],
                dst_ref=comms[ci].at[:, pl.ds(RS_OFF[2], 256)],
                send_sem=send_sems.at[ci * SEMS_PER_CHAIN + 2],
                recv_sem=recv_sems.at[ci * SEMS_PER_CHAIN + 2],
                device_id=(partner[ax],),
                device_id_type=pl.DeviceIdType.MESH,
            )
            rdma.start()
            return rdma

        def finish_leaf(ci):
            r0, rl, _ = CHAINS[ci]
            out_ref[pl.ds(r0, rl), dscol(cstart[ci], 256)] = (
                out_ref[pl.ds(r0, rl), dscol(cstart[ci], 256)]
                + comms[ci][:, RS_OFF[2]:RS_OFF[2] + 256]
            )

        def start_ag(ci, j):
            r0, rl, band = CHAINS[ci]
            cur = 128 << j
            ax = ORDERS[band][2 - j]
            rdma = pltpu.make_async_remote_copy(
                src_ref=out_ref.at[pl.ds(r0, rl), dscol(cstart[ci], cur)],
                dst_ref=out_ref.at[pl.ds(r0, rl), dscol(cstart[ci], cur)],
                send_sem=send_sems.at[ci * SEMS_PER_CHAIN + 2 + j],
                recv_sem=recv_sems.at[ci * SEMS_PER_CHAIN + 2 + j],
                device_id=(partner[ax],),
                device_id_type=pl.DeviceIdType.MESH,
            )
            rdma.start()
            return rdma

        def merge_ag(ci, j):
            _, _, band = CHAINS[ci]
            cur = 128 << j
            cstart[ci] = cstart[ci] - bit[ORDERS[band][2 - j]] * cur

        rdmas = [None] * len(CHAINS)
        for band, (br0, brl) in enumerate(BAND_ROWS):
            out_ref[br0:br0 + brl, :] = jnp.dot(
                x_ref[br0:br0 + brl, :], w_ref[...],
                preferred_element_type=jnp.float32,
            )
            for ci, (_, _, b) in enumerate(CHAINS):
                if b == band:
                    rdmas[ci] = start_rs(ci, 0)

        for ci in range(len(CHAINS)):
            rdmas[ci].wait()
            finish_rs(ci, 0)
            rdmas[ci] = start_rs(ci, 1)
        for ci in range(len(CHAINS)):
            rdmas[ci].wait()
            finish_rs(ci, 1)
            rdmas[ci] = start_leaf(ci)
        for ci in range(len(CHAINS)):
            rdmas[ci].wait()
            finish_leaf(ci)
            rdmas[ci] = start_ag(ci, 1)
        for ci in range(len(CHAINS)):
            rdmas[ci].wait()
            merge_ag(ci, 1)
            rdmas[ci] = start_ag(ci, 2)
        for ci in range(len(CHAINS)):
            rdmas[ci].wait()
            merge_ag(ci, 2)

    n_sems = SEMS_PER_CHAIN * len(CHAINS)
    return pl.pallas_call(
        body,
        out_shape=jax.ShapeDtypeStruct((M, N), jnp.float32),
        in_specs=[
            pl.BlockSpec(memory_space=pltpu.VMEM),
            pl.BlockSpec(memory_space=pltpu.VMEM),
        ],
        out_specs=pl.BlockSpec(memory_space=pltpu.VMEM),
        scratch_shapes=[
            pltpu.VMEM((rl, 1024), jnp.float32) for (_, rl, _) in CHAINS
        ] + [
            pltpu.SemaphoreType.DMA((n_sems,)),
            pltpu.SemaphoreType.DMA((n_sems,)),
        ],
        compiler_params=pltpu.CompilerParams(collective_id=0),
    )(x, w_mat)
